# baseline (speedup 1.0000x reference)
import functools

import jax
import jax.numpy as jnp
from jax.experimental import pallas as pl
from jax.experimental.pallas import tpu as pltpu

DIM_C = 32
D = 64
SEQ = 65536
BLK = 4096


def _body(z_ref, m_ref, bias_ref, w2_ref, b2_ref, out_ref):
    zb = z_ref[:, 0, :]
    a = jnp.dot(zb, m_ref[...], preferred_element_type=jnp.float32)
    a = a + bias_ref[...]
    a1 = a[:, :DIM_C]
    v1 = jnp.where(a1 > 0, a1, jnp.exp(jnp.minimum(a1, 0.0)) - 1.0)
    a2 = a[:, DIM_C:]
    g = jnp.maximum(a2, 0.0) + jnp.log(1.0 + jnp.exp(-jnp.abs(a2)))
    dc = jnp.dot(v1, w2_ref[...], preferred_element_type=jnp.float32)
    dc = dc + b2_ref[...]
    c = zb[:, :DIM_C]
    h = zb[:, DIM_C:]
    num = jnp.sum(dc * c, axis=1, keepdims=True)
    den = jnp.sum(c * c, axis=1, keepdims=True)
    dc = dc - (num / den) * c
    out = jnp.concatenate([dc, -g * h], axis=1)
    out_ref[...] = out[:, None, :]


def kernel(t, z, F_cur_W, F_cur_b, F_out_W, F_out_b, G_W, G_b):
    m = jnp.zeros((D, D), jnp.float32)
    m = m.at[:, :DIM_C].set(F_cur_W.T)
    m = m.at[:DIM_C, DIM_C:].set(G_W.T)
    bias = jnp.concatenate([F_cur_b, G_b])[None, :]
    w2 = F_out_W[:, :DIM_C].T
    b2 = F_out_b[None, :]

    grid = (SEQ // BLK,)
    out = pl.pallas_call(
        _body,
        grid=grid,
        in_specs=[
            pl.BlockSpec((BLK, 1, D), lambda i: (i, 0, 0)),
            pl.BlockSpec((D, D), lambda i: (0, 0)),
            pl.BlockSpec((1, D), lambda i: (0, 0)),
            pl.BlockSpec((DIM_C, DIM_C), lambda i: (0, 0)),
            pl.BlockSpec((1, DIM_C), lambda i: (0, 0)),
        ],
        out_specs=pl.BlockSpec((BLK, 1, D), lambda i: (i, 0, 0)),
        out_shape=jax.ShapeDtypeStruct((SEQ, 1, D), jnp.float32),
    )(z, m, bias, w2, b2)
    return out


# speedup vs baseline: 2.6457x; 2.6457x over previous
import jax
import jax.numpy as jnp
from jax.experimental import pallas as pl

DIM_C = 32
D = 64
SEQ = 65536
BLK = 4096


def _body(z_ref, m_ref, bias_ref, m2_ref, b2_ref, r1_ref, r2_ref, out_ref):
    zb = z_ref[...]
    lane = jax.lax.broadcasted_iota(jnp.int32, (BLK, D), 1)
    is_c = lane < DIM_C
    a = jnp.dot(zb, m_ref[...], preferred_element_type=jnp.float32)
    a = a + bias_ref[...]
    e = jnp.exp(jnp.where(is_c, jnp.minimum(a, 0.0), -jnp.abs(a)))
    celu = jnp.where(a > 0, a, e - 1.0)
    sp = jnp.maximum(a, 0.0) + jnp.log(1.0 + e)
    act = jnp.where(is_c, celu, sp)
    b = jnp.dot(act, m2_ref[...], preferred_element_type=jnp.float32)
    b = b + b2_ref[...]
    t = b * zb
    s = zb * zb
    nb = jnp.dot(t, r1_ref[...], preferred_element_type=jnp.float32)
    db = jnp.dot(s, r2_ref[...], preferred_element_type=jnp.float32)
    mh = jnp.where(is_c, 0.0, 1.0)
    out_ref[...] = b - (nb / db + mh * act) * zb


def kernel(t, z, F_cur_W, F_cur_b, F_out_W, F_out_b, G_W, G_b):
    m = jnp.zeros((D, D), jnp.float32)
    m = m.at[:, :DIM_C].set(F_cur_W.T)
    m = m.at[:DIM_C, DIM_C:].set(G_W.T)
    bias = jnp.concatenate([F_cur_b, G_b])[None, :]
    m2 = jnp.zeros((D, D), jnp.float32)
    m2 = m2.at[:DIM_C, :DIM_C].set(F_out_W[:, :DIM_C].T)
    b2 = jnp.concatenate([F_out_b, jnp.zeros((DIM_C,), jnp.float32)])[None, :]
    r1 = jnp.zeros((D, D), jnp.float32)
    r1 = r1.at[:DIM_C, :DIM_C].set(1.0)
    r2 = jnp.zeros((D, D), jnp.float32)
    r2 = r2.at[:DIM_C, :].set(1.0)

    zr = z.reshape(SEQ, D)
    grid = (SEQ // BLK,)
    full = lambda i: (0, 0)
    out = pl.pallas_call(
        _body,
        grid=grid,
        in_specs=[
            pl.BlockSpec((BLK, D), lambda i: (i, 0)),
            pl.BlockSpec((D, D), full),
            pl.BlockSpec((1, D), full),
            pl.BlockSpec((D, D), full),
            pl.BlockSpec((1, D), full),
            pl.BlockSpec((D, D), full),
            pl.BlockSpec((D, D), full),
        ],
        out_specs=pl.BlockSpec((BLK, D), lambda i: (i, 0)),
        out_shape=jax.ShapeDtypeStruct((SEQ, D), jnp.float32),
    )(zr, m, bias, m2, b2, r1, r2)
    return out.reshape(SEQ, 1, D)
